# Initial kernel scaffold; baseline (speedup 1.0000x reference)
import jax
import jax.numpy as jnp
from jax.experimental import pallas as pl


def kernel(node_feats, edge_feats, edge_index, W_node, b_node, W_edge, b_edge, W_msg, b_msg, W_logit, b_logit, W_ih, b_ih, W_hh, b_hh):
    raise NotImplementedError("write your pallas kernel here")


# speedup vs baseline: 6.8990x; 6.8990x over previous
import functools

import jax
import jax.numpy as jnp
from jax import lax
from jax.experimental import pallas as pl
from jax.experimental.pallas import tpu as pltpu
from jax.experimental.pallas import tpu_sc as plsc

N = 10000
E = 320000
DF = 128
DE = 16
NH = 128
EH = 128
CT = 128
TW = 144
CH = 128
NCHUNK = E // CH
NWORK = 32
NPAD = 10240
RPT = NPAD // 16


def _prep_body(nf_ref, wn_ref, bn_ref, we1_ref, w1p_ref, b1p_ref,
               hv_ref, t_ref):
    nf = nf_ref[...]
    hv = nf @ wn_ref[...] + bn_ref[...]
    hv = jnp.where(hv > 0, hv, 0.01 * hv)
    hv_ref[...] = hv
    p = nf @ we1_ref[...]
    a = hv @ w1p_ref[...] + b1p_ref[...]
    t_ref[...] = jnp.concatenate([p, a], axis=1)


def _q_body(ef_ref, we2_ref, be_ref, q_ref):
    q_ref[...] = ef_ref[...] @ we2_ref[...] + be_ref[...]


def _post_body(a_ref, hv_ref, wm_ref, bm_ref, wih_t_ref, bih_ref,
               whh_t_ref, bhh_ref, out_ref):
    acc = a_ref[0, :, :] + a_ref[1, :, :]
    s = acc[:, :NH]
    den = acc[:, NH:NH + 1]
    mask = den > 0
    cbar = jnp.where(mask, s / jnp.where(mask, den, 1.0), 0.0)
    cpre = cbar @ wm_ref[...] + jnp.where(mask, 1.0, 0.0) * bm_ref[...]
    c = jnp.where(cpre > 0, cpre, jnp.exp(jnp.minimum(cpre, 0.0)) - 1.0)
    hv = hv_ref[...]
    gi = c @ wih_t_ref[...] + bih_ref[...]
    gh = hv @ whh_t_ref[...] + bhh_ref[...]
    r = jax.nn.sigmoid(gi[:, :NH] + gh[:, :NH])
    z = jax.nn.sigmoid(gi[:, NH:2 * NH] + gh[:, NH:2 * NH])
    n = jnp.tanh(gi[:, 2 * NH:] + r * gh[:, 2 * NH:])
    h_new = (1.0 - z) * n + z * hv
    out_ref[...] = jnp.maximum(h_new, 0.0)


def _sc_edge_body(t_hbm, q_hbm, src_hbm, dst_hbm, zeros_hbm, w2_hbm, out_hbm,
                  src_v, dst_v, t_v, q_v, w2_v, a_sh, sem):
    cid = lax.axis_index("c")
    sid = lax.axis_index("s")
    wid = sid * 2 + cid

    pltpu.sync_copy(zeros_hbm.at[pl.ds(sid * RPT, RPT)],
                    a_sh.at[pl.ds(sid * RPT, RPT)])
    pltpu.sync_copy(w2_hbm, w2_v)
    w2r = [w2_v[pl.ds(j * 16, 16)] for j in range(8)]
    plsc.subcore_barrier()

    def chunk_body(k, carry):
        c = wid + k * NWORK

        @pl.when(c < NCHUNK)
        def _():
            pltpu.sync_copy(src_hbm.at[c], src_v)
            pltpu.sync_copy(dst_hbm.at[c], dst_v)
            pltpu.async_copy(t_hbm.at[src_v], t_v, sem).wait()
            pltpu.sync_copy(q_hbm.at[pl.ds(c * CH, CH)], q_v)

            def edge_body(e, carry2):
                acc = t_v[e, pl.ds(NH, 16)]
                hs = []
                for j in range(8):
                    g = t_v[e, pl.ds(j * 16, 16)] + q_v[e, pl.ds(j * 16, 16)]
                    h = jnp.where(g > 0, g, 0.01 * g)
                    hs.append(h)
                    acc = acc + h * w2r[j]
                lanes = lax.iota(jnp.int32, 16)
                dnums = lax.GatherDimensionNumbers(
                    offset_dims=(), collapsed_slice_dims=(0,),
                    start_index_map=(0,))
                for sh in (8, 4, 2, 1):
                    perm = jnp.bitwise_xor(lanes, sh)
                    acc = acc + lax.gather(
                        acc, perm[:, None], dnums, (1,),
                        unique_indices=True,
                        mode=lax.GatherScatterMode.PROMISE_IN_BOUNDS)
                tb = acc
                lv = jnp.where(tb > 0, tb, 0.01 * tb)
                exv = jnp.exp(lv)
                for j in range(8):
                    t_v[e, pl.ds(j * 16, 16)] = hs[j] * exv
                t_v[e, pl.ds(NH, 16)] = exv
                return carry2

            lax.fori_loop(0, CH, edge_body, 0)
            pltpu.sync_copy(t_v, a_sh.at[dst_v], add=True)

        return carry

    lax.fori_loop(0, (NCHUNK + NWORK - 1) // NWORK, chunk_body, 0)

    plsc.subcore_barrier()
    pltpu.sync_copy(a_sh.at[pl.ds(sid * RPT, RPT)],
                    out_hbm.at[cid, pl.ds(sid * RPT, RPT)])


@jax.jit
def kernel(node_feats, edge_feats, edge_index, W_node, b_node, W_edge, b_edge,
           W_msg, b_msg, W_logit, b_logit, W_ih, b_ih, W_hh, b_hh):
    we1 = W_edge[:DF]
    we2 = W_edge[DF:]
    w1p = jnp.pad(W_logit[:NH], ((0, 0), (0, 15)))
    b1p = jnp.pad(b_logit, (0, 15))
    w2 = W_logit[NH:, 0]
    wih_t = W_ih.T
    whh_t = W_hh.T
    src2d = edge_index[0].reshape(NCHUNK, CH)
    dst2d = edge_index[1].reshape(NCHUNK, CH)
    zeros = jnp.zeros((NPAD, TW), jnp.float32)

    hv, t_tab = pl.pallas_call(
        _prep_body,
        out_shape=(
            jax.ShapeDtypeStruct((N, NH), jnp.float32),
            jax.ShapeDtypeStruct((N, TW), jnp.float32),
        ),
    )(node_feats, W_node, b_node, we1, w1p, b1p)

    bq = 8000
    q = pl.pallas_call(
        _q_body,
        grid=(E // bq,),
        in_specs=[
            pl.BlockSpec((bq, DE), lambda i: (i, 0)),
            pl.BlockSpec((DE, EH), lambda i: (0, 0)),
            pl.BlockSpec((EH,), lambda i: (0,)),
        ],
        out_specs=pl.BlockSpec((bq, EH), lambda i: (i, 0)),
        out_shape=jax.ShapeDtypeStruct((E, EH), jnp.float32),
    )(edge_feats, we2, b_edge)

    a_out = pl.kernel(
        _sc_edge_body,
        out_type=jax.ShapeDtypeStruct((2, NPAD, TW), jnp.float32),
        mesh=plsc.VectorSubcoreMesh(core_axis_name="c", subcore_axis_name="s",
                                    num_cores=2, num_subcores=16),
        compiler_params=pltpu.CompilerParams(use_tc_tiling_on_sc=False),
        scratch_types=[
            pltpu.VMEM((CH,), jnp.int32),
            pltpu.VMEM((CH,), jnp.int32),
            pltpu.VMEM((CH, TW), jnp.float32),
            pltpu.VMEM((CH, EH), jnp.float32),
            pltpu.VMEM((NH,), jnp.float32),
            pltpu.VMEM_SHARED((NPAD, TW), jnp.float32),
            pltpu.SemaphoreType.DMA,
        ],
    )(t_tab, q, src2d, dst2d, zeros, w2)
    a_out = a_out[:, :N]

    h_out = pl.pallas_call(
        _post_body,
        out_shape=jax.ShapeDtypeStruct((N, NH), jnp.float32),
    )(a_out, hv, W_msg, b_msg, wih_t, b_ih, whh_t, b_hh)

    return (h_out, edge_feats)
